# baseline (speedup 1.0000x reference)
import functools

import jax
import jax.numpy as jnp
from jax import lax
from jax.experimental import pallas as pl
from jax.experimental.pallas import tpu as pltpu
from jax.experimental.pallas import tpu_sc as plsc

HIDDEN = 4096
N_EXPERTS = 64
TOP_K = 8
BT = 1024

NC, NS, L = 2, 16, 16
NW = NC * NS


def _softmax_block(x_ref, w_ref, p_ref):
    logits = jax.lax.dot_general(
        x_ref[...].astype(jnp.bfloat16), w_ref[...].astype(jnp.bfloat16),
        dimension_numbers=(((1,), (1,)), ((), ())),
        preferred_element_type=jnp.float32,
    )
    m = jnp.max(logits, axis=1, keepdims=True)
    e = jnp.exp(logits - m)
    p_ref[...] = e / jnp.sum(e, axis=1, keepdims=True)


def _tc_softmax(x, W):
    tokens = x.shape[0]
    return pl.pallas_call(
        _softmax_block,
        grid=(tokens // BT,),
        in_specs=[
            pl.BlockSpec((BT, HIDDEN), lambda i: (i, 0)),
            pl.BlockSpec((N_EXPERTS, HIDDEN), lambda i: (0, 0)),
        ],
        out_specs=pl.BlockSpec((BT, N_EXPERTS), lambda i: (i, 0)),
        out_shape=jax.ShapeDtypeStruct((tokens, N_EXPERTS), jnp.float32),
        compiler_params=pltpu.CompilerParams(
            dimension_semantics=("arbitrary",),
            vmem_limit_bytes=100 * 1024 * 1024,
        ),
    )(x, W)


def _make_sc_topk(tokens):
    tok_w = tokens // NW
    ch = 128
    mesh = plsc.VectorSubcoreMesh(core_axis_name="c", subcore_axis_name="s")

    @functools.partial(
        pl.kernel, mesh=mesh,
        compiler_params=pltpu.CompilerParams(needs_layout_passes=False),
        out_type=[
            jax.ShapeDtypeStruct((tokens, TOP_K), jnp.float32),
            jax.ShapeDtypeStruct((tokens, TOP_K), jnp.int32),
        ],
        scratch_types=[
            pltpu.VMEM((ch, N_EXPERTS), jnp.float32),
            pltpu.VMEM((ch, TOP_K), jnp.float32),
            pltpu.VMEM((ch, TOP_K), jnp.int32),
        ],
    )
    def sc_topk(p_hbm, vals_hbm, idx_hbm, p_v, vals_v, idx_v):
        wid = lax.axis_index("s") * NC + lax.axis_index("c")
        base = wid * tok_w
        lane = lax.iota(jnp.int32, L)

        def chunk(c, carry):
            off = base + c * ch
            pltpu.sync_copy(p_hbm.at[pl.ds(off, ch)], p_v)

            def group(g, carry2):
                toks = lane + g * L
                tops = [jnp.zeros((L,), jnp.int32) for _ in range(TOP_K)]
                for j in range(N_EXPERTS):
                    v = plsc.load_gather(
                        p_v, [toks, jnp.full((L,), j, jnp.int32)])
                    cur = ((plsc.bitcast(v, jnp.int32) & jnp.int32(~63))
                           | jnp.int32(63 - j))
                    for i in range(TOP_K):
                        hi = jnp.maximum(tops[i], cur)
                        cur = jnp.minimum(tops[i], cur)
                        tops[i] = hi
                for i in range(TOP_K):
                    jx = jnp.int32(63) - (tops[i] & jnp.int32(63))
                    val = plsc.load_gather(p_v, [toks, jx])
                    col = jnp.full((L,), i, jnp.int32)
                    plsc.store_scatter(vals_v, [toks, col], val)
                    plsc.store_scatter(idx_v, [toks, col], jx)
                return carry2

            lax.fori_loop(0, ch // L, group, 0)
            pltpu.sync_copy(vals_v, vals_hbm.at[pl.ds(off, ch)])
            pltpu.sync_copy(idx_v, idx_hbm.at[pl.ds(off, ch)])
            return carry

        lax.fori_loop(0, tok_w // ch, chunk, 0)

    return sc_topk


@jax.jit
def kernel(x, W):
    p = _tc_softmax(x, W)
    vals, idx = _make_sc_topk(x.shape[0])(p)
    return vals, idx


# speedup vs baseline: 1.3906x; 1.3906x over previous
import functools

import jax
import jax.numpy as jnp
from jax.experimental import pallas as pl
from jax.experimental.pallas import tpu as pltpu

HIDDEN = 4096
N_EXPERTS = 64
TOP_K = 8
BT = 1024


def _gate_block(x_ref, w_ref, vals_ref, idx_ref):
    logits = jax.lax.dot_general(
        x_ref[...].astype(jnp.bfloat16), w_ref[...].astype(jnp.bfloat16),
        dimension_numbers=(((1,), (1,)), ((), ())),
        preferred_element_type=jnp.float32,
    )
    m = jnp.max(logits, axis=1, keepdims=True)
    e = jnp.exp(logits - m)
    s = jnp.sum(e, axis=1, keepdims=True)

    iota = jax.lax.broadcasted_iota(jnp.int32, e.shape, 1).astype(jnp.float32)
    for k in range(TOP_K):
        v = jnp.max(e, axis=1, keepdims=True)
        cand = jnp.where(e == v, iota, float(N_EXPERTS))
        ix = jnp.min(cand, axis=1, keepdims=True)
        vals_ref[:, k] = (v / s)[:, 0]
        idx_ref[:, k] = ix[:, 0].astype(jnp.int32)
        e = jnp.where(iota == ix, -1.0, e)


@jax.jit
def kernel(x, W):
    tokens = x.shape[0]
    grid = (pl.cdiv(tokens, BT),)
    vals, idx = pl.pallas_call(
        _gate_block,
        grid=grid,
        in_specs=[
            pl.BlockSpec((BT, HIDDEN), lambda i: (i, 0)),
            pl.BlockSpec((N_EXPERTS, HIDDEN), lambda i: (0, 0)),
        ],
        out_specs=[
            pl.BlockSpec((BT, TOP_K), lambda i: (i, 0)),
            pl.BlockSpec((BT, TOP_K), lambda i: (i, 0)),
        ],
        out_shape=[
            jax.ShapeDtypeStruct((tokens, TOP_K), jnp.float32),
            jax.ShapeDtypeStruct((tokens, TOP_K), jnp.int32),
        ],
        compiler_params=pltpu.CompilerParams(
            dimension_semantics=("arbitrary",),
            vmem_limit_bytes=100 * 1024 * 1024,
        ),
    )(x, W)
    return vals, idx
